# baseline (speedup 1.0000x reference)
import functools

import jax
import jax.numpy as jnp
from jax import lax
from jax.experimental import pallas as pl
from jax.experimental.pallas import tpu as pltpu
from jax.experimental.pallas import tpu_sc as plsc

H = 240
W = 320
HW = H * W
NQ = 64
N_PAD = 784 * 128
ROWS = 784

NSUB = 16
PTS_PER_TILE = N_PAD // NSUB
GROUPS = PTS_PER_TILE // 16
KMAP_SIZE = HW + 64
PIX_PER_TILE = HW // 32
PGROUPS = PIX_PER_TILE // 16
GCHUNK = 120
NCHUNK = PIX_PER_TILE // GCHUNK
HALF = HW // 2

KMAP2 = HALF + 32
PIXOFF = KMAP2
KEYOFF = PIXOFF + PTS_PER_TILE
VWORDS = 50976
MOFF = 0
AOFF = MOFF + 8 * PIX_PER_TILE
WOFF = AOFF + PIX_PER_TILE
GOFF = WOFF + PIX_PER_TILE


def _project_body(pc0_ref, pc1_ref, pc2_ref, cf_ref, k_ref, s_ref,
                  pix_ref, key_ref, zb_ref):
    pc0 = pc0_ref[...]
    pc1 = pc1_ref[...]
    pc2 = pc2_ref[...]
    cf = cf_ref[...]
    fx = k_ref[0, 0]; cx = k_ref[0, 2]; fy = k_ref[1, 1]; cy = k_ref[1, 2]
    znear = s_ref[0, 0]; zfar = s_ref[0, 1]; cth = s_ref[0, 2]

    absz = jnp.abs(pc2)
    xc = pc0 * fx / absz + cx
    yc = pc1 * fy / absz + cy
    xi = jnp.round(xc).astype(jnp.int32)
    yi = jnp.round(yc).astype(jnp.int32)
    oob = ((xi < 0) | (xi >= W) | (yi < 0) | (yi >= H)
           | (absz < znear) | (absz > zfar) | (cf <= cth))
    invlo = 1.0 / jnp.where(oob, 1e-10, absz)
    invhi = 1.0 / jnp.where(oob, 1e10, absz)
    dmin = jnp.min(invlo)
    dmax = jnp.max(invhi)
    do = ((invhi - dmin) / (dmax - dmin) * (NQ - 1)).astype(jnp.int32)

    idx = (lax.broadcasted_iota(jnp.int32, (ROWS, 128), 0) * 128
           + lax.broadcasted_iota(jnp.int32, (ROWS, 128), 1))
    key = jnp.where(oob, -1, do * 131072 + idx)
    pix = jnp.where(oob, HW + (idx & 63), (H - 1 - yi) * W + xi)
    pix_ref[...] = pix
    key_ref[...] = key
    zb_ref[...] = lax.bitcast_convert_type(absz, jnp.int32)


def _project(pc0, pc1, pc2, cf, k, scal):
    return pl.pallas_call(
        _project_body,
        out_shape=[
            jax.ShapeDtypeStruct((ROWS, 128), jnp.int32),
            jax.ShapeDtypeStruct((ROWS, 128), jnp.int32),
            jax.ShapeDtypeStruct((ROWS, 128), jnp.int32),
        ],
    )(pc0, pc1, pc2, cf, k, scal)


def _maxpool3(x):
    rp = jnp.pad(x, ((1, 1), (0, 0)), constant_values=-jnp.inf)
    rm = jnp.maximum(jnp.maximum(rp[0:H, :], rp[1:H + 1, :]), rp[2:H + 2, :])
    cp = jnp.pad(rm, ((0, 0), (1, 1)), constant_values=-jnp.inf)
    return jnp.maximum(jnp.maximum(cp[:, 0:W], cp[:, 1:W + 1]), cp[:, 2:W + 2])


def _holefill_body(d_ref, c_ref, r_ref, g_ref, b_ref,
                   do_ref, co_ref, rgb_ref):
    d = d_ref[...]
    c = c_ref[...]
    r = r_ref[...]
    g = g_ref[...]
    b = b_ref[...]
    for _ in range(2):
        p = jnp.pad(d, 1)
        cols = [p[:, dj:dj + W] for dj in range(3)]

        def s(di, dj):
            return cols[dj][di:di + H, :]

        cs = [col[0:H, :] + col[1:H + 1, :] + col[2:H + 2, :] for col in cols]
        nsum = cs[0] + cs[1] + cs[2]
        tofill = (nsum > 0) & (d <= 0)
        o0 = cs[0]
        o1 = s(2, 0) + s(2, 1) + s(2, 2)
        o2 = cs[2]
        o3 = s(0, 0) + s(0, 1) + s(0, 2)
        o4 = s(1, 0) + s(2, 0) + s(2, 1)
        o5 = s(1, 2) + s(2, 1) + s(2, 2)
        o6 = s(0, 1) + s(0, 2) + s(1, 2)
        o7 = s(0, 0) + s(0, 1) + s(1, 0)
        prod = ((o0 * o1) * (o2 * o3)) * ((o4 * o5) * (o6 * o7))
        fill = (jnp.abs(prod) > 1e-10) & tofill
        d, c, r, g, b = (jnp.where(fill, _maxpool3(v), v)
                         for v in (d, c, r, g, b))
    do_ref[0, 0] = d
    co_ref[0, 0] = c
    rgb_ref[0] = r
    rgb_ref[1] = g
    rgb_ref[2] = b


def _holefill(d, c, r, g, b):
    return pl.pallas_call(
        _holefill_body,
        out_shape=[
            jax.ShapeDtypeStruct((1, 1, H, W), jnp.float32),
            jax.ShapeDtypeStruct((1, 1, H, W), jnp.float32),
            jax.ShapeDtypeStruct((3, H, W), jnp.float32),
        ],
    )(d, c, r, g, b)


def _sc_body(pix_hbm, key_hbm, packed_hbm,
             oz, oc, orr, og, ob,
             vbuf, grows, shared, sem0, sem1):
    c = lax.axis_index("c")
    s = lax.axis_index("s")
    wid = c * NSUB + s
    pbase = s * PTS_PER_TILE

    cp_p = pltpu.async_copy(pix_hbm.at[pl.ds(pbase, PTS_PER_TILE)],
                            vbuf.at[pl.ds(PIXOFF, PTS_PER_TILE)], sem0)
    cp_k = pltpu.async_copy(key_hbm.at[pl.ds(pbase, PTS_PER_TILE)],
                            vbuf.at[pl.ds(KEYOFF, PTS_PER_TILE)], sem0)
    neg1 = jnp.full((16,), -1, jnp.int32)

    def init_body(i, carry):
        vbuf[pl.ds(i * 16, 16)] = neg1
        return carry

    with jax.named_scope("sc_init"):
        lax.fori_loop(0, KMAP2 // 16, init_body, jnp.int32(0), unroll=16)
        cp_p.wait()
        cp_k.wait()

    _dnums = lax.GatherDimensionNumbers(
        offset_dims=(), collapsed_slice_dims=(0,), start_index_map=(0,))
    lane = lax.iota(jnp.int32, 16)

    def _rot(x, d):
        idx = jnp.bitwise_and(lane + d, 15).reshape(16, 1)
        return lax.gather(x, idx, _dnums, (1,),
                          mode=lax.GatherScatterMode.PROMISE_IN_BOUNDS)

    half_base = c * HALF

    def scan_body(g, carry):
        pg = vbuf[pl.ds(PIXOFF + g * 16, 16)]
        kv0 = vbuf[pl.ds(KEYOFF + g * 16, 16)]
        t = pg - half_base
        pv = jnp.where((t >= 0) & (t < HALF), t, HALF + lane)
        terms = [kv0]
        for dist in range(1, 16):
            pr = _rot(pv, dist)
            kr = _rot(kv0, dist)
            terms.append(jnp.where(pr == pv, kr, kv0))
        while len(terms) > 1:
            terms = [jnp.maximum(a, b) for a, b in zip(terms[::2], terms[1::2])] \
                + ([terms[-1]] if len(terms) % 2 else [])
        kv = terms[0]
        cur = plsc.load_gather(vbuf, [pv])
        plsc.store_scatter(vbuf, [pv], jnp.maximum(cur, kv))
        return carry

    with jax.named_scope("sc_scan"):
        lax.fori_loop(0, GROUPS, scan_body, jnp.int32(0), unroll=4)

    with jax.named_scope("sc_publish"):
        pltpu.sync_copy(vbuf.at[pl.ds(0, HALF)],
                        shared.at[pl.ds(pl.multiple_of(s * HALF, 8), HALF)])
        plsc.subcore_barrier()

    wbase = pl.multiple_of(wid * PIX_PER_TILE, 8)
    for phase in range(2):
        for m in range(8):
            src_off = pl.multiple_of((phase * 8 + m) * HALF + s * PIX_PER_TILE, 8)
            pltpu.sync_copy(shared.at[pl.ds(src_off, PIX_PER_TILE)],
                            vbuf.at[pl.ds(MOFF + m * PIX_PER_TILE, PIX_PER_TILE)])

        def merge_body(g, carry, phase=phase):
            if phase == 0:
                acc = vbuf[pl.ds(MOFF + g * 16, 16)]
                rows = range(1, 8)
            else:
                acc = vbuf[pl.ds(AOFF + g * 16, 16)]
                rows = range(8)
            for m in rows:
                acc = jnp.maximum(
                    acc, vbuf[pl.ds(MOFF + m * PIX_PER_TILE + g * 16, 16)])
            vbuf[pl.ds(AOFF + g * 16, 16)] = acc
            return carry

        with jax.named_scope("sc_merge"):
            lax.fori_loop(0, PGROUPS, merge_body, jnp.int32(0), unroll=2)

    def wi_body(g, carry):
        fin = vbuf[pl.ds(AOFF + g * 16, 16)]
        dummy = wbase + (g % 64) * 16 + lane
        vbuf[pl.ds(WOFF + g * 16, 16)] = jnp.where(
            fin >= 0, jnp.bitwise_and(fin, 131071), dummy)
        return carry

    lax.fori_loop(0, PGROUPS, wi_body, jnp.int32(0), unroll=4)

    with jax.named_scope("sc_gather"):
        copies = []
        for j in range(NCHUNK):
            idx = vbuf.at[pl.ds(WOFF + j * GCHUNK, GCHUNK)]
            copies.append(pltpu.async_copy(
                packed_hbm.at[idx], grows.at[pl.ds(j * GCHUNK, GCHUNK)], sem1))
        for cp in copies:
            cp.wait()

    zero = jnp.zeros((16,), jnp.int32)

    def fin_body(g, carry):
        valid = vbuf[pl.ds(AOFF + g * 16, 16)] >= 0
        rvec = g * 16 + lane
        for t in range(5):
            col = lane * 0 + t
            v = plsc.load_gather(grows, [rvec, col])
            vbuf[pl.ds(GOFF + t * PIX_PER_TILE + g * 16, 16)] = \
                jnp.where(valid, v, zero)
        return carry

    with jax.named_scope("sc_fin"):
        lax.fori_loop(0, PGROUPS, fin_body, jnp.int32(0), unroll=4)
        for t, dst in enumerate((oz, oc, orr, og, ob)):
            pltpu.sync_copy(vbuf.at[pl.ds(GOFF + t * PIX_PER_TILE, PIX_PER_TILE)],
                            dst.at[pl.ds(wbase, PIX_PER_TILE)])


def _scatter_gather_sc(pix, key, packed):
    fn = pl.kernel(
        _sc_body,
        out_type=[jax.ShapeDtypeStruct((HW,), jnp.int32)] * 5,
        mesh=plsc.VectorSubcoreMesh(core_axis_name="c", subcore_axis_name="s"),
        compiler_params=pltpu.CompilerParams(needs_layout_passes=False,
                                             use_tc_tiling_on_sc=False),
        scratch_types=[
            pltpu.VMEM((VWORDS,), jnp.int32),
            pltpu.VMEM((PIX_PER_TILE, 8), jnp.int32),
            pltpu.VMEM_SHARED((NSUB * HALF,), jnp.int32),
            pltpu.SemaphoreType.DMA,
            pltpu.SemaphoreType.DMA,
        ],
    )
    return fn(pix.reshape(N_PAD), key.reshape(N_PAD), packed)


def kernel(pp, conf, pose_w2c, K, h, w, znear=0.1, zfar=1000.0, conf_thresh=0.0):
    n = pp.shape[2]
    pad = N_PAD - n

    def prep(a):
        return jnp.pad(a, (0, pad)).reshape(ROWS, 128)

    pc = jnp.matmul(pose_w2c, pp[:, :4, :])
    cf = prep(conf[0])
    scal = jnp.asarray([znear, zfar, conf_thresh], jnp.float32).reshape(1, 3)
    k = K[0].astype(jnp.float32)

    pix, key, zb = _project(prep(pc[0, 0]), prep(pc[0, 1]), prep(pc[0, 2]),
                            cf, k, scal)
    zf = lax.bitcast_convert_type(zb.reshape(N_PAD)[:n], jnp.float32)
    vals = jnp.stack([zf, conf[0], pp[0, 4], pp[0, 5], pp[0, 6]], axis=1)
    packed = lax.bitcast_convert_type(
        jnp.pad(vals, ((0, 0), (0, 3))), jnp.int32)

    zm, cm, rm, gm, bm = _scatter_gather_sc(pix, key, packed)

    def as_img(a):
        return lax.bitcast_convert_type(a, jnp.float32).reshape(H, W)

    depths, confs, rgb3 = _holefill(as_img(zm), as_img(cm), as_img(rm),
                                    as_img(gm), as_img(bm))
    return (depths, confs, rgb3.reshape(1, 3, H, W))


# speedup vs baseline: 2.1418x; 2.1418x over previous
import functools

import jax
import jax.numpy as jnp
from jax import lax
from jax.experimental import pallas as pl
from jax.experimental.pallas import tpu as pltpu
from jax.experimental.pallas import tpu_sc as plsc

H = 240
W = 320
HW = H * W
NQ = 64
N_PAD = 784 * 128
ROWS = 784

NSUB = 16
PTS_PER_TILE = N_PAD // NSUB
GROUPS = PTS_PER_TILE // 16
KMAP_SIZE = HW + 64
PIX_PER_TILE = HW // 32
PGROUPS = PIX_PER_TILE // 16
GCHUNK = 120
NCHUNK = PIX_PER_TILE // GCHUNK
HALF = HW // 2

KMAP2 = HALF + 32
PIXOFF = KMAP2
KEYOFF = PIXOFF + PTS_PER_TILE
VWORDS = 50976
MOFF = 0
AOFF = MOFF + 8 * PIX_PER_TILE
WOFF = AOFF + PIX_PER_TILE
GOFF = WOFF + PIX_PER_TILE


def _project_body(pc0_ref, pc1_ref, pc2_ref, cf_ref, k_ref, s_ref,
                  pix_ref, key_ref, zb_ref):
    pc0 = pc0_ref[...]
    pc1 = pc1_ref[...]
    pc2 = pc2_ref[...]
    cf = cf_ref[...]
    fx = k_ref[0, 0]; cx = k_ref[0, 2]; fy = k_ref[1, 1]; cy = k_ref[1, 2]
    znear = s_ref[0, 0]; zfar = s_ref[0, 1]; cth = s_ref[0, 2]

    absz = jnp.abs(pc2)
    xc = pc0 * fx / absz + cx
    yc = pc1 * fy / absz + cy
    xi = jnp.round(xc).astype(jnp.int32)
    yi = jnp.round(yc).astype(jnp.int32)
    oob = ((xi < 0) | (xi >= W) | (yi < 0) | (yi >= H)
           | (absz < znear) | (absz > zfar) | (cf <= cth))
    invlo = 1.0 / jnp.where(oob, 1e-10, absz)
    invhi = 1.0 / jnp.where(oob, 1e10, absz)
    dmin = jnp.min(invlo)
    dmax = jnp.max(invhi)
    do = ((invhi - dmin) / (dmax - dmin) * (NQ - 1)).astype(jnp.int32)

    idx = (lax.broadcasted_iota(jnp.int32, (ROWS, 128), 0) * 128
           + lax.broadcasted_iota(jnp.int32, (ROWS, 128), 1))
    key = jnp.where(oob, -1, do * 131072 + idx)
    pix = jnp.where(oob, HW + (idx & 63), (H - 1 - yi) * W + xi)
    pix_ref[...] = pix
    key_ref[...] = key
    zb_ref[...] = lax.bitcast_convert_type(absz, jnp.int32)


def _project(pc0, pc1, pc2, cf, k, scal):
    return pl.pallas_call(
        _project_body,
        out_shape=[
            jax.ShapeDtypeStruct((ROWS, 128), jnp.int32),
            jax.ShapeDtypeStruct((ROWS, 128), jnp.int32),
            jax.ShapeDtypeStruct((ROWS, 128), jnp.int32),
        ],
    )(pc0, pc1, pc2, cf, k, scal)


def _maxpool3(x):
    rp = jnp.pad(x, ((1, 1), (0, 0)), constant_values=-jnp.inf)
    rm = jnp.maximum(jnp.maximum(rp[0:H, :], rp[1:H + 1, :]), rp[2:H + 2, :])
    cp = jnp.pad(rm, ((0, 0), (1, 1)), constant_values=-jnp.inf)
    return jnp.maximum(jnp.maximum(cp[:, 0:W], cp[:, 1:W + 1]), cp[:, 2:W + 2])


def _holefill_body(d_ref, c_ref, r_ref, g_ref, b_ref,
                   do_ref, co_ref, rgb_ref):
    d = d_ref[...]
    c = c_ref[...]
    r = r_ref[...]
    g = g_ref[...]
    b = b_ref[...]
    for _ in range(2):
        p = jnp.pad(d, 1)
        cols = [p[:, dj:dj + W] for dj in range(3)]

        def s(di, dj):
            return cols[dj][di:di + H, :]

        cs = [col[0:H, :] + col[1:H + 1, :] + col[2:H + 2, :] for col in cols]
        nsum = cs[0] + cs[1] + cs[2]
        tofill = (nsum > 0) & (d <= 0)
        o0 = cs[0]
        o1 = s(2, 0) + s(2, 1) + s(2, 2)
        o2 = cs[2]
        o3 = s(0, 0) + s(0, 1) + s(0, 2)
        o4 = s(1, 0) + s(2, 0) + s(2, 1)
        o5 = s(1, 2) + s(2, 1) + s(2, 2)
        o6 = s(0, 1) + s(0, 2) + s(1, 2)
        o7 = s(0, 0) + s(0, 1) + s(1, 0)
        prod = ((o0 * o1) * (o2 * o3)) * ((o4 * o5) * (o6 * o7))
        fill = (jnp.abs(prod) > 1e-10) & tofill
        d, c, r, g, b = (jnp.where(fill, _maxpool3(v), v)
                         for v in (d, c, r, g, b))
    do_ref[0, 0] = d
    co_ref[0, 0] = c
    rgb_ref[0] = r
    rgb_ref[1] = g
    rgb_ref[2] = b


def _holefill(d, c, r, g, b):
    return pl.pallas_call(
        _holefill_body,
        out_shape=[
            jax.ShapeDtypeStruct((1, 1, H, W), jnp.float32),
            jax.ShapeDtypeStruct((1, 1, H, W), jnp.float32),
            jax.ShapeDtypeStruct((3, H, W), jnp.float32),
        ],
    )(d, c, r, g, b)


def _sc_body(pix_hbm, key_hbm, zb_hbm, cb_hbm, rb_hbm, gb_hbm, bb_hbm,
             oz, oc, orr, og, ob,
             vbuf, shared, shz, shc, shr, shg, shb, sem0, sem1, sem2):
    c = lax.axis_index("c")
    s = lax.axis_index("s")
    wid = c * NSUB + s
    pbase = s * PTS_PER_TILE

    cp_p = pltpu.async_copy(pix_hbm.at[pl.ds(pbase, PTS_PER_TILE)],
                            vbuf.at[pl.ds(PIXOFF, PTS_PER_TILE)], sem0)
    cp_k = pltpu.async_copy(key_hbm.at[pl.ds(pbase, PTS_PER_TILE)],
                            vbuf.at[pl.ds(KEYOFF, PTS_PER_TILE)], sem0)
    stage = [pltpu.async_copy(tbl.at[pl.ds(pbase, PTS_PER_TILE)],
                              sh.at[pl.ds(pbase, PTS_PER_TILE)], sem2)
             for tbl, sh in ((zb_hbm, shz), (cb_hbm, shc), (rb_hbm, shr),
                             (gb_hbm, shg), (bb_hbm, shb))]
    neg1 = jnp.full((16,), -1, jnp.int32)

    def init_body(i, carry):
        vbuf[pl.ds(i * 16, 16)] = neg1
        return carry

    with jax.named_scope("sc_init"):
        lax.fori_loop(0, KMAP2 // 16, init_body, jnp.int32(0), unroll=16)
        cp_p.wait()
        cp_k.wait()

    _dnums = lax.GatherDimensionNumbers(
        offset_dims=(), collapsed_slice_dims=(0,), start_index_map=(0,))
    lane = lax.iota(jnp.int32, 16)

    def _rot(x, d):
        idx = jnp.bitwise_and(lane + d, 15).reshape(16, 1)
        return lax.gather(x, idx, _dnums, (1,),
                          mode=lax.GatherScatterMode.PROMISE_IN_BOUNDS)

    half_base = c * HALF

    def scan_body(g, carry):
        pg = vbuf[pl.ds(PIXOFF + g * 16, 16)]
        kv0 = vbuf[pl.ds(KEYOFF + g * 16, 16)]
        t = pg - half_base
        pv = jnp.where((t >= 0) & (t < HALF), t, HALF + lane)
        terms = [kv0]
        for dist in range(1, 16):
            pr = _rot(pv, dist)
            kr = _rot(kv0, dist)
            terms.append(jnp.where(pr == pv, kr, kv0))
        while len(terms) > 1:
            terms = [jnp.maximum(a, b) for a, b in zip(terms[::2], terms[1::2])] \
                + ([terms[-1]] if len(terms) % 2 else [])
        kv = terms[0]
        cur = plsc.load_gather(vbuf, [pv])
        plsc.store_scatter(vbuf, [pv], jnp.maximum(cur, kv))
        return carry

    with jax.named_scope("sc_scan"):
        lax.fori_loop(0, GROUPS, scan_body, jnp.int32(0), unroll=4)

    with jax.named_scope("sc_publish"):
        pltpu.sync_copy(vbuf.at[pl.ds(0, HALF)],
                        shared.at[pl.ds(pl.multiple_of(s * HALF, 8), HALF)])
        for cp in stage:
            cp.wait()
        plsc.subcore_barrier()

    wbase = pl.multiple_of(wid * PIX_PER_TILE, 8)
    for phase in range(2):
        for m in range(8):
            src_off = pl.multiple_of((phase * 8 + m) * HALF + s * PIX_PER_TILE, 8)
            pltpu.sync_copy(shared.at[pl.ds(src_off, PIX_PER_TILE)],
                            vbuf.at[pl.ds(MOFF + m * PIX_PER_TILE, PIX_PER_TILE)])

        def merge_body(g, carry, phase=phase):
            if phase == 0:
                acc = vbuf[pl.ds(MOFF + g * 16, 16)]
                rows = range(1, 8)
            else:
                acc = vbuf[pl.ds(AOFF + g * 16, 16)]
                rows = range(8)
            for m in rows:
                acc = jnp.maximum(
                    acc, vbuf[pl.ds(MOFF + m * PIX_PER_TILE + g * 16, 16)])
            vbuf[pl.ds(AOFF + g * 16, 16)] = acc
            return carry

        with jax.named_scope("sc_merge"):
            lax.fori_loop(0, PGROUPS, merge_body, jnp.int32(0), unroll=2)

    def wi_body(g, carry):
        fin = vbuf[pl.ds(AOFF + g * 16, 16)]
        dummy = wbase + (g % 64) * 16 + lane
        vbuf[pl.ds(WOFF + g * 16, 16)] = jnp.where(
            fin >= 0, jnp.bitwise_and(fin, 131071), dummy)
        return carry

    lax.fori_loop(0, PGROUPS, wi_body, jnp.int32(0), unroll=4)

    with jax.named_scope("sc_gather"):
        copies = []
        for j in range(NCHUNK):
            idx = vbuf.at[pl.ds(WOFF + j * GCHUNK, GCHUNK)]
            for t, tbl in enumerate((shz, shc, shr, shg, shb)):
                dst = vbuf.at[pl.ds(GOFF + t * PIX_PER_TILE + j * GCHUNK, GCHUNK)]
                copies.append(pltpu.async_copy(tbl.at[idx], dst, sem1))
        for cp in copies:
            cp.wait()

    zero = jnp.zeros((16,), jnp.int32)

    def fin_body(g, carry):
        valid = vbuf[pl.ds(AOFF + g * 16, 16)] >= 0
        for t in range(5):
            off = GOFF + t * PIX_PER_TILE + g * 16
            vbuf[pl.ds(off, 16)] = jnp.where(valid, vbuf[pl.ds(off, 16)], zero)
        return carry

    with jax.named_scope("sc_fin"):
        lax.fori_loop(0, PGROUPS, fin_body, jnp.int32(0), unroll=4)
        for t, dst in enumerate((oz, oc, orr, og, ob)):
            pltpu.sync_copy(vbuf.at[pl.ds(GOFF + t * PIX_PER_TILE, PIX_PER_TILE)],
                            dst.at[pl.ds(wbase, PIX_PER_TILE)])


def _scatter_gather_sc(pix, key, zb, cb, rb, gb, bb):
    fn = pl.kernel(
        _sc_body,
        out_type=[jax.ShapeDtypeStruct((HW,), jnp.int32)] * 5,
        mesh=plsc.VectorSubcoreMesh(core_axis_name="c", subcore_axis_name="s"),
        compiler_params=pltpu.CompilerParams(needs_layout_passes=False),
        scratch_types=[
            pltpu.VMEM((VWORDS,), jnp.int32),
            pltpu.VMEM_SHARED((NSUB * HALF,), jnp.int32),
            pltpu.VMEM_SHARED((N_PAD,), jnp.int32),
            pltpu.VMEM_SHARED((N_PAD,), jnp.int32),
            pltpu.VMEM_SHARED((N_PAD,), jnp.int32),
            pltpu.VMEM_SHARED((N_PAD,), jnp.int32),
            pltpu.VMEM_SHARED((N_PAD,), jnp.int32),
            pltpu.SemaphoreType.DMA,
            pltpu.SemaphoreType.DMA,
            pltpu.SemaphoreType.DMA,
        ],
    )
    return fn(pix.reshape(N_PAD), key.reshape(N_PAD), zb.reshape(N_PAD),
              cb, rb, gb, bb)


def kernel(pp, conf, pose_w2c, K, h, w, znear=0.1, zfar=1000.0, conf_thresh=0.0):
    n = pp.shape[2]
    pad = N_PAD - n

    def prep(a):
        return jnp.pad(a, (0, pad)).reshape(ROWS, 128)

    pc = jnp.matmul(pose_w2c, pp[:, :4, :])
    cf = prep(conf[0])
    scal = jnp.asarray([znear, zfar, conf_thresh], jnp.float32).reshape(1, 3)
    k = K[0].astype(jnp.float32)

    pix, key, zb = _project(prep(pc[0, 0]), prep(pc[0, 1]), prep(pc[0, 2]),
                            cf, k, scal)
    cb = lax.bitcast_convert_type(cf, jnp.int32).reshape(N_PAD)
    rb = lax.bitcast_convert_type(prep(pp[0, 4]), jnp.int32).reshape(N_PAD)
    gb = lax.bitcast_convert_type(prep(pp[0, 5]), jnp.int32).reshape(N_PAD)
    bb = lax.bitcast_convert_type(prep(pp[0, 6]), jnp.int32).reshape(N_PAD)

    zm, cm, rm, gm, bm = _scatter_gather_sc(pix, key, zb, cb, rb, gb, bb)

    def as_img(a):
        return lax.bitcast_convert_type(a, jnp.float32).reshape(H, W)

    depths, confs, rgb3 = _holefill(as_img(zm), as_img(cm), as_img(rm),
                                    as_img(gm), as_img(bm))
    return (depths, confs, rgb3.reshape(1, 3, H, W))


# speedup vs baseline: 2.2126x; 1.0330x over previous
import functools

import jax
import jax.numpy as jnp
from jax import lax
from jax.experimental import pallas as pl
from jax.experimental.pallas import tpu as pltpu
from jax.experimental.pallas import tpu_sc as plsc

H = 240
W = 320
HW = H * W
NQ = 64
N_PAD = 784 * 128
ROWS = 784

NSUB = 16
PTS_PER_TILE = N_PAD // NSUB
GROUPS = PTS_PER_TILE // 16
KMAP_SIZE = HW + 64
PIX_PER_TILE = HW // 32
PGROUPS = PIX_PER_TILE // 16
GCHUNK = 120
NCHUNK = PIX_PER_TILE // GCHUNK
HALF = HW // 2

KMAP2 = HALF + 32
PIXOFF = KMAP2
KEYOFF = PIXOFF + PTS_PER_TILE
VWORDS = 55200
MOFF = 0
AOFF = MOFF + 16 * PIX_PER_TILE
WOFF = AOFF + PIX_PER_TILE
GOFF = WOFF + PIX_PER_TILE


def _project_body(pc0_ref, pc1_ref, pc2_ref, cf_ref, k_ref, s_ref,
                  pix_ref, key_ref, zb_ref):
    pc0 = pc0_ref[...]
    pc1 = pc1_ref[...]
    pc2 = pc2_ref[...]
    cf = cf_ref[...]
    fx = k_ref[0, 0]; cx = k_ref[0, 2]; fy = k_ref[1, 1]; cy = k_ref[1, 2]
    znear = s_ref[0, 0]; zfar = s_ref[0, 1]; cth = s_ref[0, 2]

    absz = jnp.abs(pc2)
    xc = pc0 * fx / absz + cx
    yc = pc1 * fy / absz + cy
    xi = jnp.round(xc).astype(jnp.int32)
    yi = jnp.round(yc).astype(jnp.int32)
    oob = ((xi < 0) | (xi >= W) | (yi < 0) | (yi >= H)
           | (absz < znear) | (absz > zfar) | (cf <= cth))
    invlo = 1.0 / jnp.where(oob, 1e-10, absz)
    invhi = 1.0 / jnp.where(oob, 1e10, absz)
    dmin = jnp.min(invlo)
    dmax = jnp.max(invhi)
    do = ((invhi - dmin) / (dmax - dmin) * (NQ - 1)).astype(jnp.int32)

    idx = (lax.broadcasted_iota(jnp.int32, (ROWS, 128), 0) * 128
           + lax.broadcasted_iota(jnp.int32, (ROWS, 128), 1))
    key = jnp.where(oob, -1, do * 131072 + idx)
    pix = jnp.where(oob, HW + (idx & 63), (H - 1 - yi) * W + xi)
    pix_ref[...] = pix
    key_ref[...] = key
    zb_ref[...] = lax.bitcast_convert_type(absz, jnp.int32)


def _project(pc0, pc1, pc2, cf, k, scal):
    return pl.pallas_call(
        _project_body,
        out_shape=[
            jax.ShapeDtypeStruct((ROWS, 128), jnp.int32),
            jax.ShapeDtypeStruct((ROWS, 128), jnp.int32),
            jax.ShapeDtypeStruct((ROWS, 128), jnp.int32),
        ],
    )(pc0, pc1, pc2, cf, k, scal)


def _maxpool3(x):
    rp = jnp.pad(x, ((1, 1), (0, 0)), constant_values=-jnp.inf)
    rm = jnp.maximum(jnp.maximum(rp[0:H, :], rp[1:H + 1, :]), rp[2:H + 2, :])
    cp = jnp.pad(rm, ((0, 0), (1, 1)), constant_values=-jnp.inf)
    return jnp.maximum(jnp.maximum(cp[:, 0:W], cp[:, 1:W + 1]), cp[:, 2:W + 2])


def _holefill_body(d_ref, c_ref, r_ref, g_ref, b_ref,
                   do_ref, co_ref, rgb_ref):
    d = d_ref[...]
    c = c_ref[...]
    r = r_ref[...]
    g = g_ref[...]
    b = b_ref[...]
    for _ in range(2):
        p = jnp.pad(d, 1)
        cols = [p[:, dj:dj + W] for dj in range(3)]

        def s(di, dj):
            return cols[dj][di:di + H, :]

        cs = [col[0:H, :] + col[1:H + 1, :] + col[2:H + 2, :] for col in cols]
        nsum = cs[0] + cs[1] + cs[2]
        tofill = (nsum > 0) & (d <= 0)
        o0 = cs[0]
        o1 = s(2, 0) + s(2, 1) + s(2, 2)
        o2 = cs[2]
        o3 = s(0, 0) + s(0, 1) + s(0, 2)
        o4 = s(1, 0) + s(2, 0) + s(2, 1)
        o5 = s(1, 2) + s(2, 1) + s(2, 2)
        o6 = s(0, 1) + s(0, 2) + s(1, 2)
        o7 = s(0, 0) + s(0, 1) + s(1, 0)
        prod = ((o0 * o1) * (o2 * o3)) * ((o4 * o5) * (o6 * o7))
        fill = (jnp.abs(prod) > 1e-10) & tofill
        d, c, r, g, b = (jnp.where(fill, _maxpool3(v), v)
                         for v in (d, c, r, g, b))
    do_ref[0, 0] = d
    co_ref[0, 0] = c
    rgb_ref[0] = r
    rgb_ref[1] = g
    rgb_ref[2] = b


def _holefill(d, c, r, g, b):
    return pl.pallas_call(
        _holefill_body,
        out_shape=[
            jax.ShapeDtypeStruct((1, 1, H, W), jnp.float32),
            jax.ShapeDtypeStruct((1, 1, H, W), jnp.float32),
            jax.ShapeDtypeStruct((3, H, W), jnp.float32),
        ],
    )(d, c, r, g, b)


def _sc_body(pix_hbm, key_hbm, zb_hbm, cb_hbm, rb_hbm, gb_hbm, bb_hbm,
             oz, oc, orr, og, ob,
             vbuf, shared, shz, shc, shr, shg, shb, sem0, sem1, sem2):
    c = lax.axis_index("c")
    s = lax.axis_index("s")
    wid = c * NSUB + s
    pbase = s * PTS_PER_TILE

    cp_p = pltpu.async_copy(pix_hbm.at[pl.ds(pbase, PTS_PER_TILE)],
                            vbuf.at[pl.ds(PIXOFF, PTS_PER_TILE)], sem0)
    cp_k = pltpu.async_copy(key_hbm.at[pl.ds(pbase, PTS_PER_TILE)],
                            vbuf.at[pl.ds(KEYOFF, PTS_PER_TILE)], sem0)
    stage = [pltpu.async_copy(tbl.at[pl.ds(pbase, PTS_PER_TILE)],
                              sh.at[pl.ds(pbase, PTS_PER_TILE)], sem2)
             for tbl, sh in ((zb_hbm, shz), (cb_hbm, shc), (rb_hbm, shr),
                             (gb_hbm, shg), (bb_hbm, shb))]
    neg1 = jnp.full((16,), -1, jnp.int32)

    def init_body(i, carry):
        vbuf[pl.ds(i * 16, 16)] = neg1
        return carry

    with jax.named_scope("sc_init"):
        lax.fori_loop(0, KMAP2 // 16, init_body, jnp.int32(0), unroll=16)
        cp_p.wait()
        cp_k.wait()

    _dnums = lax.GatherDimensionNumbers(
        offset_dims=(), collapsed_slice_dims=(0,), start_index_map=(0,))
    lane = lax.iota(jnp.int32, 16)

    def _rot(x, d):
        idx = jnp.bitwise_and(lane + d, 15).reshape(16, 1)
        return lax.gather(x, idx, _dnums, (1,),
                          mode=lax.GatherScatterMode.PROMISE_IN_BOUNDS)

    half_base = c * HALF

    def scan_body(g, carry):
        pg = vbuf[pl.ds(PIXOFF + g * 16, 16)]
        kv0 = vbuf[pl.ds(KEYOFF + g * 16, 16)]
        t = pg - half_base
        pv = jnp.where((t >= 0) & (t < HALF), t, HALF + lane)
        terms = [kv0]
        for dist in range(1, 16):
            pr = _rot(pv, dist)
            kr = _rot(kv0, dist)
            terms.append(jnp.where(pr == pv, kr, kv0))
        while len(terms) > 1:
            terms = [jnp.maximum(a, b) for a, b in zip(terms[::2], terms[1::2])] \
                + ([terms[-1]] if len(terms) % 2 else [])
        kv = terms[0]
        cur = plsc.load_gather(vbuf, [pv])
        plsc.store_scatter(vbuf, [pv], jnp.maximum(cur, kv))
        return carry

    with jax.named_scope("sc_scan"):
        lax.fori_loop(0, GROUPS, scan_body, jnp.int32(0), unroll=8)

    with jax.named_scope("sc_publish"):
        pltpu.sync_copy(vbuf.at[pl.ds(0, HALF)],
                        shared.at[pl.ds(pl.multiple_of(s * HALF, 8), HALF)])
        for cp in stage:
            cp.wait()
        plsc.subcore_barrier()

    wbase = pl.multiple_of(wid * PIX_PER_TILE, 8)
    with jax.named_scope("sc_merge"):
        mcopies = []
        for m in range(16):
            src_off = pl.multiple_of(m * HALF + s * PIX_PER_TILE, 8)
            mcopies.append(pltpu.async_copy(
                shared.at[pl.ds(src_off, PIX_PER_TILE)],
                vbuf.at[pl.ds(MOFF + m * PIX_PER_TILE, PIX_PER_TILE)], sem0))
        for cp in mcopies:
            cp.wait()

        def merge_body(g, carry):
            terms = [vbuf[pl.ds(MOFF + m * PIX_PER_TILE + g * 16, 16)]
                     for m in range(16)]
            while len(terms) > 1:
                terms = [jnp.maximum(a, b)
                         for a, b in zip(terms[::2], terms[1::2])]
            vbuf[pl.ds(AOFF + g * 16, 16)] = terms[0]
            return carry

        lax.fori_loop(0, PGROUPS, merge_body, jnp.int32(0), unroll=2)

    def wi_body(g, carry):
        fin = vbuf[pl.ds(AOFF + g * 16, 16)]
        dummy = wbase + (g % 64) * 16 + lane
        vbuf[pl.ds(WOFF + g * 16, 16)] = jnp.where(
            fin >= 0, jnp.bitwise_and(fin, 131071), dummy)
        return carry

    lax.fori_loop(0, PGROUPS, wi_body, jnp.int32(0), unroll=4)

    with jax.named_scope("sc_gather"):
        copies = []
        for j in range(NCHUNK):
            idx = vbuf.at[pl.ds(WOFF + j * GCHUNK, GCHUNK)]
            for t, tbl in enumerate((shz, shc, shr, shg, shb)):
                dst = vbuf.at[pl.ds(GOFF + t * PIX_PER_TILE + j * GCHUNK, GCHUNK)]
                copies.append(pltpu.async_copy(tbl.at[idx], dst, sem1))
        for cp in copies:
            cp.wait()

    zero = jnp.zeros((16,), jnp.int32)

    def fin_body(g, carry):
        valid = vbuf[pl.ds(AOFF + g * 16, 16)] >= 0
        for t in range(5):
            off = GOFF + t * PIX_PER_TILE + g * 16
            vbuf[pl.ds(off, 16)] = jnp.where(valid, vbuf[pl.ds(off, 16)], zero)
        return carry

    with jax.named_scope("sc_fin"):
        lax.fori_loop(0, PGROUPS, fin_body, jnp.int32(0), unroll=4)
        for t, dst in enumerate((oz, oc, orr, og, ob)):
            pltpu.sync_copy(vbuf.at[pl.ds(GOFF + t * PIX_PER_TILE, PIX_PER_TILE)],
                            dst.at[pl.ds(wbase, PIX_PER_TILE)])


def _scatter_gather_sc(pix, key, zb, cb, rb, gb, bb):
    fn = pl.kernel(
        _sc_body,
        out_type=[jax.ShapeDtypeStruct((HW,), jnp.int32)] * 5,
        mesh=plsc.VectorSubcoreMesh(core_axis_name="c", subcore_axis_name="s"),
        compiler_params=pltpu.CompilerParams(needs_layout_passes=False),
        scratch_types=[
            pltpu.VMEM((VWORDS,), jnp.int32),
            pltpu.VMEM_SHARED((NSUB * HALF,), jnp.int32),
            pltpu.VMEM_SHARED((N_PAD,), jnp.int32),
            pltpu.VMEM_SHARED((N_PAD,), jnp.int32),
            pltpu.VMEM_SHARED((N_PAD,), jnp.int32),
            pltpu.VMEM_SHARED((N_PAD,), jnp.int32),
            pltpu.VMEM_SHARED((N_PAD,), jnp.int32),
            pltpu.SemaphoreType.DMA,
            pltpu.SemaphoreType.DMA,
            pltpu.SemaphoreType.DMA,
        ],
    )
    return fn(pix.reshape(N_PAD), key.reshape(N_PAD), zb.reshape(N_PAD),
              cb, rb, gb, bb)


def kernel(pp, conf, pose_w2c, K, h, w, znear=0.1, zfar=1000.0, conf_thresh=0.0):
    n = pp.shape[2]
    pad = N_PAD - n

    def prep(a):
        return jnp.pad(a, (0, pad)).reshape(ROWS, 128)

    pc = jnp.matmul(pose_w2c, pp[:, :4, :])
    cf = prep(conf[0])
    scal = jnp.asarray([znear, zfar, conf_thresh], jnp.float32).reshape(1, 3)
    k = K[0].astype(jnp.float32)

    pix, key, zb = _project(prep(pc[0, 0]), prep(pc[0, 1]), prep(pc[0, 2]),
                            cf, k, scal)
    cb = lax.bitcast_convert_type(cf, jnp.int32).reshape(N_PAD)
    rb = lax.bitcast_convert_type(prep(pp[0, 4]), jnp.int32).reshape(N_PAD)
    gb = lax.bitcast_convert_type(prep(pp[0, 5]), jnp.int32).reshape(N_PAD)
    bb = lax.bitcast_convert_type(prep(pp[0, 6]), jnp.int32).reshape(N_PAD)

    zm, cm, rm, gm, bm = _scatter_gather_sc(pix, key, zb, cb, rb, gb, bb)

    def as_img(a):
        return lax.bitcast_convert_type(a, jnp.float32).reshape(H, W)

    depths, confs, rgb3 = _holefill(as_img(zm), as_img(cm), as_img(rm),
                                    as_img(gm), as_img(bm))
    return (depths, confs, rgb3.reshape(1, 3, H, W))


# speedup vs baseline: 2.2138x; 1.0005x over previous
import jax
import jax.numpy as jnp
from jax import lax
from jax.experimental import pallas as pl
from jax.experimental.pallas import tpu as pltpu
from jax.experimental.pallas import tpu_sc as plsc

H = 240
W = 320
HW = H * W
NQ = 64
N_PAD = 784 * 128
ROWS = 784

NSUB = 16
PTS_PER_TILE = N_PAD // NSUB
GROUPS = PTS_PER_TILE // 16
PIX_PER_TILE = HW // 32
PGROUPS = PIX_PER_TILE // 16
GCHUNK = 120
NCHUNK = PIX_PER_TILE // GCHUNK
HALF = HW // 2

KMAP2 = HALF + 32
PIXOFF = KMAP2
KEYOFF = PIXOFF + PTS_PER_TILE
VWORDS = 55200
MOFF = 0
AOFF = MOFF + 16 * PIX_PER_TILE
WOFF = AOFF + PIX_PER_TILE
GOFF = WOFF + PIX_PER_TILE


def _project_body(pc0_ref, pc1_ref, pc2_ref, cf_ref, k_ref, s_ref,
                  pix_ref, key_ref, zb_ref):
    pc0 = pc0_ref[...]
    pc1 = pc1_ref[...]
    pc2 = pc2_ref[...]
    cf = cf_ref[...]
    fx = k_ref[0, 0]; cx = k_ref[0, 2]; fy = k_ref[1, 1]; cy = k_ref[1, 2]
    znear = s_ref[0, 0]; zfar = s_ref[0, 1]; cth = s_ref[0, 2]

    absz = jnp.abs(pc2)
    xc = pc0 * fx / absz + cx
    yc = pc1 * fy / absz + cy
    xi = jnp.round(xc).astype(jnp.int32)
    yi = jnp.round(yc).astype(jnp.int32)
    oob = ((xi < 0) | (xi >= W) | (yi < 0) | (yi >= H)
           | (absz < znear) | (absz > zfar) | (cf <= cth))
    invlo = 1.0 / jnp.where(oob, 1e-10, absz)
    invhi = 1.0 / jnp.where(oob, 1e10, absz)
    dmin = jnp.min(invlo)
    dmax = jnp.max(invhi)
    do = ((invhi - dmin) / (dmax - dmin) * (NQ - 1)).astype(jnp.int32)

    idx = (lax.broadcasted_iota(jnp.int32, (ROWS, 128), 0) * 128
           + lax.broadcasted_iota(jnp.int32, (ROWS, 128), 1))
    key = jnp.where(oob, -1, do * 131072 + idx)
    pix = jnp.where(oob, HW + (idx & 63), (H - 1 - yi) * W + xi)
    pix_ref[...] = pix
    key_ref[...] = key
    zb_ref[...] = lax.bitcast_convert_type(absz, jnp.int32)


def _project(pc0, pc1, pc2, cf, k, scal):
    return pl.pallas_call(
        _project_body,
        out_shape=[
            jax.ShapeDtypeStruct((ROWS, 128), jnp.int32),
            jax.ShapeDtypeStruct((ROWS, 128), jnp.int32),
            jax.ShapeDtypeStruct((ROWS, 128), jnp.int32),
        ],
    )(pc0, pc1, pc2, cf, k, scal)


def _maxpool3(x):
    rp = jnp.pad(x, ((1, 1), (0, 0)), constant_values=-jnp.inf)
    rm = jnp.maximum(jnp.maximum(rp[0:H, :], rp[1:H + 1, :]), rp[2:H + 2, :])
    cp = jnp.pad(rm, ((0, 0), (1, 1)), constant_values=-jnp.inf)
    return jnp.maximum(jnp.maximum(cp[:, 0:W], cp[:, 1:W + 1]), cp[:, 2:W + 2])


def _holefill_body(d_ref, c_ref, r_ref, g_ref, b_ref,
                   do_ref, co_ref, rgb_ref):
    d = d_ref[...]
    c = c_ref[...]
    r = r_ref[...]
    g = g_ref[...]
    b = b_ref[...]
    for _ in range(2):
        p = jnp.pad(d, 1)
        cols = [p[:, dj:dj + W] for dj in range(3)]

        def s(di, dj):
            return cols[dj][di:di + H, :]

        cs = [col[0:H, :] + col[1:H + 1, :] + col[2:H + 2, :] for col in cols]
        nsum = cs[0] + cs[1] + cs[2]
        tofill = (nsum > 0) & (d <= 0)
        o0 = cs[0]
        o1 = s(2, 0) + s(2, 1) + s(2, 2)
        o2 = cs[2]
        o3 = s(0, 0) + s(0, 1) + s(0, 2)
        o4 = s(1, 0) + s(2, 0) + s(2, 1)
        o5 = s(1, 2) + s(2, 1) + s(2, 2)
        o6 = s(0, 1) + s(0, 2) + s(1, 2)
        o7 = s(0, 0) + s(0, 1) + s(1, 0)
        prod = ((o0 * o1) * (o2 * o3)) * ((o4 * o5) * (o6 * o7))
        fill = (jnp.abs(prod) > 1e-10) & tofill
        d, c, r, g, b = (jnp.where(fill, _maxpool3(v), v)
                         for v in (d, c, r, g, b))
    do_ref[0, 0] = d
    co_ref[0, 0] = c
    rgb_ref[0] = r
    rgb_ref[1] = g
    rgb_ref[2] = b


def _holefill(d, c, r, g, b):
    return pl.pallas_call(
        _holefill_body,
        out_shape=[
            jax.ShapeDtypeStruct((1, 1, H, W), jnp.float32),
            jax.ShapeDtypeStruct((1, 1, H, W), jnp.float32),
            jax.ShapeDtypeStruct((3, H, W), jnp.float32),
        ],
    )(d, c, r, g, b)


def _sc_body(pix_hbm, key_hbm, zb_hbm, cb_hbm, rb_hbm, gb_hbm, bb_hbm,
             oz, oc, orr, og, ob,
             vbuf, shared, shz, shc, shr, shg, shb, sem0, sem1, sem2):
    c = lax.axis_index("c")
    s = lax.axis_index("s")
    wid = c * NSUB + s
    pbase = s * PTS_PER_TILE

    cp_p = pltpu.async_copy(pix_hbm.at[pl.ds(pbase, PTS_PER_TILE)],
                            vbuf.at[pl.ds(PIXOFF, PTS_PER_TILE)], sem0)
    cp_k = pltpu.async_copy(key_hbm.at[pl.ds(pbase, PTS_PER_TILE)],
                            vbuf.at[pl.ds(KEYOFF, PTS_PER_TILE)], sem0)
    stage = [pltpu.async_copy(tbl.at[pl.ds(pbase, PTS_PER_TILE)],
                              sh.at[pl.ds(pbase, PTS_PER_TILE)], sem2)
             for tbl, sh in ((zb_hbm, shz), (cb_hbm, shc), (rb_hbm, shr),
                             (gb_hbm, shg), (bb_hbm, shb))]
    neg1 = jnp.full((16,), -1, jnp.int32)

    def init_body(i, carry):
        vbuf[pl.ds(i * 16, 16)] = neg1
        return carry

    with jax.named_scope("sc_init"):
        lax.fori_loop(0, KMAP2 // 16, init_body, jnp.int32(0), unroll=16)
        cp_p.wait()
        cp_k.wait()

    _dnums = lax.GatherDimensionNumbers(
        offset_dims=(), collapsed_slice_dims=(0,), start_index_map=(0,))
    lane = lax.iota(jnp.int32, 16)

    def _rot(x, d):
        idx = jnp.bitwise_and(lane + d, 15).reshape(16, 1)
        return lax.gather(x, idx, _dnums, (1,),
                          mode=lax.GatherScatterMode.PROMISE_IN_BOUNDS)

    half_base = c * HALF

    def scan_body(g, carry):
        pg = vbuf[pl.ds(PIXOFF + g * 16, 16)]
        kv0 = vbuf[pl.ds(KEYOFF + g * 16, 16)]
        t = pg - half_base
        pv = jnp.where((t >= 0) & (t < HALF), t, HALF + lane)
        terms = [kv0]
        for dist in range(1, 16):
            pr = _rot(pv, dist)
            kr = _rot(kv0, dist)
            terms.append(jnp.where(pr == pv, kr, kv0))
        while len(terms) > 1:
            terms = [jnp.maximum(a, b) for a, b in zip(terms[::2], terms[1::2])] \
                + ([terms[-1]] if len(terms) % 2 else [])
        kv = terms[0]
        cur = plsc.load_gather(vbuf, [pv])
        plsc.store_scatter(vbuf, [pv], jnp.maximum(cur, kv))
        return carry

    with jax.named_scope("sc_scan"):
        lax.fori_loop(0, GROUPS, scan_body, jnp.int32(0), unroll=8)

    with jax.named_scope("sc_publish"):
        pltpu.sync_copy(vbuf.at[pl.ds(0, HALF)],
                        shared.at[pl.ds(pl.multiple_of(s * HALF, 8), HALF)])
        for cp in stage:
            cp.wait()
        plsc.subcore_barrier()

    wbase = pl.multiple_of(wid * PIX_PER_TILE, 8)
    with jax.named_scope("sc_merge"):
        mcopies = []
        for m in range(16):
            src_off = pl.multiple_of(m * HALF + s * PIX_PER_TILE, 8)
            mcopies.append(pltpu.async_copy(
                shared.at[pl.ds(src_off, PIX_PER_TILE)],
                vbuf.at[pl.ds(MOFF + m * PIX_PER_TILE, PIX_PER_TILE)], sem0))
        for cp in mcopies:
            cp.wait()

        def merge_body(g, carry):
            terms = [vbuf[pl.ds(MOFF + m * PIX_PER_TILE + g * 16, 16)]
                     for m in range(16)]
            while len(terms) > 1:
                terms = [jnp.maximum(a, b)
                         for a, b in zip(terms[::2], terms[1::2])]
            vbuf[pl.ds(AOFF + g * 16, 16)] = terms[0]
            return carry

        lax.fori_loop(0, PGROUPS, merge_body, jnp.int32(0), unroll=2)

    def wi_body(g, carry):
        fin = vbuf[pl.ds(AOFF + g * 16, 16)]
        dummy = wbase + (g % 64) * 16 + lane
        vbuf[pl.ds(WOFF + g * 16, 16)] = jnp.where(
            fin >= 0, jnp.bitwise_and(fin, 131071), dummy)
        return carry

    lax.fori_loop(0, PGROUPS, wi_body, jnp.int32(0), unroll=4)

    with jax.named_scope("sc_gather"):
        copies = []
        for j in range(NCHUNK):
            idx = vbuf.at[pl.ds(WOFF + j * GCHUNK, GCHUNK)]
            for t, tbl in enumerate((shz, shc, shr, shg, shb)):
                dst = vbuf.at[pl.ds(GOFF + t * PIX_PER_TILE + j * GCHUNK, GCHUNK)]
                copies.append(pltpu.async_copy(tbl.at[idx], dst, sem1))
        for cp in copies:
            cp.wait()

    zero = jnp.zeros((16,), jnp.int32)

    def fin_body(g, carry):
        valid = vbuf[pl.ds(AOFF + g * 16, 16)] >= 0
        for t in range(5):
            off = GOFF + t * PIX_PER_TILE + g * 16
            vbuf[pl.ds(off, 16)] = jnp.where(valid, vbuf[pl.ds(off, 16)], zero)
        return carry

    with jax.named_scope("sc_fin"):
        lax.fori_loop(0, PGROUPS, fin_body, jnp.int32(0), unroll=4)
        for t, dst in enumerate((oz, oc, orr, og, ob)):
            pltpu.sync_copy(vbuf.at[pl.ds(GOFF + t * PIX_PER_TILE, PIX_PER_TILE)],
                            dst.at[pl.ds(wbase, PIX_PER_TILE)])


def _scatter_gather_sc(pix, key, zb, cb, rb, gb, bb):
    fn = pl.kernel(
        _sc_body,
        out_type=[jax.ShapeDtypeStruct((HW,), jnp.int32)] * 5,
        mesh=plsc.VectorSubcoreMesh(core_axis_name="c", subcore_axis_name="s"),
        compiler_params=pltpu.CompilerParams(needs_layout_passes=False),
        scratch_types=[
            pltpu.VMEM((VWORDS,), jnp.int32),
            pltpu.VMEM_SHARED((NSUB * HALF,), jnp.int32),
            pltpu.VMEM_SHARED((N_PAD,), jnp.int32),
            pltpu.VMEM_SHARED((N_PAD,), jnp.int32),
            pltpu.VMEM_SHARED((N_PAD,), jnp.int32),
            pltpu.VMEM_SHARED((N_PAD,), jnp.int32),
            pltpu.VMEM_SHARED((N_PAD,), jnp.int32),
            pltpu.SemaphoreType.DMA,
            pltpu.SemaphoreType.DMA,
            pltpu.SemaphoreType.DMA,
        ],
    )
    return fn(pix.reshape(N_PAD), key.reshape(N_PAD), zb.reshape(N_PAD),
              cb, rb, gb, bb)


def kernel(pp, conf, pose_w2c, K, h, w, znear=0.1, zfar=1000.0, conf_thresh=0.0):
    n = pp.shape[2]
    pad = N_PAD - n

    def prep(a):
        return jnp.pad(a, (0, pad)).reshape(ROWS, 128)

    pc = jnp.matmul(pose_w2c, pp[:, :4, :])
    cf = prep(conf[0])
    scal = jnp.asarray([znear, zfar, conf_thresh], jnp.float32).reshape(1, 3)
    k = K[0].astype(jnp.float32)

    pix, key, zb = _project(prep(pc[0, 0]), prep(pc[0, 1]), prep(pc[0, 2]),
                            cf, k, scal)
    cb = lax.bitcast_convert_type(cf, jnp.int32).reshape(N_PAD)
    rb = lax.bitcast_convert_type(prep(pp[0, 4]), jnp.int32).reshape(N_PAD)
    gb = lax.bitcast_convert_type(prep(pp[0, 5]), jnp.int32).reshape(N_PAD)
    bb = lax.bitcast_convert_type(prep(pp[0, 6]), jnp.int32).reshape(N_PAD)

    zm, cm, rm, gm, bm = _scatter_gather_sc(pix, key, zb, cb, rb, gb, bb)

    def as_img(a):
        return lax.bitcast_convert_type(a, jnp.float32).reshape(H, W)

    depths, confs, rgb3 = _holefill(as_img(zm), as_img(cm), as_img(rm),
                                    as_img(gm), as_img(bm))
    return (depths, confs, rgb3.reshape(1, 3, H, W))
